# baseline (speedup 1.0000x reference)
import functools

import jax
import jax.numpy as jnp
from jax.experimental import pallas as pl
from jax.experimental.pallas import tpu as pltpu

_TEMP = 0.05
_SHIFT = 20.0
_CHUNK = 512


def _ce_kernel(n_valid, n_rows, x_ref, t_ref, f_ref, out_ref,
               xn_ref, s_ref, tl_ref):
    c = pl.program_id(0)
    nc = pl.num_programs(0)

    @pl.when(c == 0)
    def _init():
        x = x_ref[...]
        norm = jnp.sqrt(jnp.sum(x * x, axis=1, keepdims=True))
        xn_ref[...] = x / (jnp.maximum(norm, 1e-12) * _TEMP)
        s_ref[...] = jnp.zeros_like(s_ref)
        tl_ref[...] = jnp.zeros_like(tl_ref)

    logits = jax.lax.dot_general(
        xn_ref[...], f_ref[...], (((1,), (1,)), ((), ())),
        preferred_element_type=jnp.float32)
    cols = c * _CHUNK + jax.lax.broadcasted_iota(jnp.int32, logits.shape, 1)
    ev = jnp.where(cols < n_valid, jnp.exp(logits - _SHIFT), 0.0)
    s_ref[...] += jnp.sum(ev, axis=1, keepdims=True)
    tmask = cols == t_ref[...]
    tl_ref[...] += jnp.sum(jnp.where(tmask, logits, 0.0), axis=1, keepdims=True)

    @pl.when(c == nc - 1)
    def _fin():
        lse = jnp.log(s_ref[...]) + _SHIFT
        loss = jnp.sum(lse - tl_ref[...]) * (1.0 / n_rows)
        out_ref[...] = loss.reshape(1, 1)


@jax.jit
def kernel(inputs, targets, cameras, features):
    b, d = inputs.shape
    n = features.shape[0]
    nc = pl.cdiv(n, _CHUNK)
    n_pad = nc * _CHUNK
    fpad = jnp.pad(features, ((0, n_pad - n), (0, 0)))
    t2 = targets.astype(jnp.int32).reshape(b, 1)
    out = pl.pallas_call(
        functools.partial(_ce_kernel, n, b),
        grid=(nc,),
        in_specs=[
            pl.BlockSpec((b, d), lambda i: (0, 0)),
            pl.BlockSpec((b, 1), lambda i: (0, 0)),
            pl.BlockSpec((_CHUNK, d), lambda i: (i, 0)),
        ],
        out_specs=pl.BlockSpec((1, 1), lambda i: (0, 0)),
        out_shape=jax.ShapeDtypeStruct((1, 1), jnp.float32),
        scratch_shapes=[
            pltpu.VMEM((b, d), jnp.float32),
            pltpu.VMEM((b, 1), jnp.float32),
            pltpu.VMEM((b, 1), jnp.float32),
        ],
        compiler_params=pltpu.CompilerParams(
            dimension_semantics=("arbitrary",)),
    )(inputs, t2, fpad)
    return out[0, 0]


# speedup vs baseline: 2.4009x; 2.4009x over previous
import functools

import jax
import jax.numpy as jnp
from jax.experimental import pallas as pl
from jax.experimental.pallas import tpu as pltpu

_TEMP = 0.05
_SHIFT = 20.0
_CHUNK = 512


def _ce_kernel(n_valid, n_rows, x_ref, t_ref, f_ref, out_ref,
               xn_ref, s_ref, tl_ref):
    c = pl.program_id(0)
    nc = pl.num_programs(0)

    @pl.when(c == 0)
    def _init():
        x = x_ref[...]
        norm = jnp.sqrt(jnp.sum(x * x, axis=1, keepdims=True))
        xn_ref[...] = (x / (jnp.maximum(norm, 1e-12) * _TEMP)).astype(
            jnp.bfloat16)
        s_ref[...] = jnp.zeros_like(s_ref)
        tl_ref[...] = jnp.zeros_like(tl_ref)

    logits = jax.lax.dot_general(
        xn_ref[...], f_ref[...], (((1,), (1,)), ((), ())),
        preferred_element_type=jnp.float32)
    cols = c * _CHUNK + jax.lax.broadcasted_iota(jnp.int32, logits.shape, 1)
    ev = jnp.exp(logits - _SHIFT)
    s_ref[...] += jnp.where(cols < n_valid, ev, 0.0)
    tl_ref[...] += jnp.where(cols == t_ref[...], logits, 0.0)

    @pl.when(c == nc - 1)
    def _fin():
        lse = jnp.log(jnp.sum(s_ref[...], axis=1, keepdims=True)) + _SHIFT
        tl = jnp.sum(tl_ref[...], axis=1, keepdims=True)
        loss = jnp.sum(lse - tl) * (1.0 / n_rows)
        out_ref[...] = loss.reshape(1, 1)


@jax.jit
def kernel(inputs, targets, cameras, features):
    b, d = inputs.shape
    n = features.shape[0]
    nc = pl.cdiv(n, _CHUNK)
    n_pad = nc * _CHUNK
    fpad = jnp.pad(features.astype(jnp.bfloat16), ((0, n_pad - n), (0, 0)))
    t2 = targets.astype(jnp.int32).reshape(b, 1)
    out = pl.pallas_call(
        functools.partial(_ce_kernel, n, b),
        grid=(nc,),
        in_specs=[
            pl.BlockSpec((b, d), lambda i: (0, 0)),
            pl.BlockSpec((b, 1), lambda i: (0, 0)),
            pl.BlockSpec((_CHUNK, d), lambda i: (i, 0)),
        ],
        out_specs=pl.BlockSpec((1, 1), lambda i: (0, 0)),
        out_shape=jax.ShapeDtypeStruct((1, 1), jnp.float32),
        scratch_shapes=[
            pltpu.VMEM((b, d), jnp.bfloat16),
            pltpu.VMEM((b, _CHUNK), jnp.float32),
            pltpu.VMEM((b, _CHUNK), jnp.float32),
        ],
        compiler_params=pltpu.CompilerParams(
            dimension_semantics=("arbitrary",)),
    )(inputs, t2, fpad)
    return out[0, 0]
